# baseline (speedup 1.0000x reference)
import functools

import jax
import jax.numpy as jnp
import numpy as np
from jax.experimental import pallas as pl
from jax.experimental.pallas import tpu as pltpu

_BF = jnp.bfloat16
_VMEM = 64 * 1024 * 1024


def _enc_conv_body(x_ref, w_ref, b_ref, o_ref, *, oh, ow):
    x = x_ref[...]
    bb = x.shape[0]
    c2 = x.shape[-1]
    taps = []
    for kh in range(4):
        qi, hp = kh // 2, kh % 2
        for dj in range(2):
            t = x[:, qi:qi + oh, hp:hp + 1, dj:dj + ow, :]
            taps.append(t.reshape(bb, oh, ow, c2))
    a = jnp.concatenate(taps, axis=-1)
    a2 = a.reshape(bb * oh * ow, a.shape[-1])
    acc = jnp.dot(a2, w_ref[...], preferred_element_type=jnp.float32)
    acc = jnp.maximum(acc + b_ref[...], 0.0)
    o_ref[...] = acc.reshape(bb, oh, ow, acc.shape[-1]).astype(o_ref.dtype)


def _enc_conv(xr, wm, b, oh, ow, bb):
    B = xr.shape[0]
    bb = min(bb, B)
    cout = wm.shape[1]
    return pl.pallas_call(
        functools.partial(_enc_conv_body, oh=oh, ow=ow),
        out_shape=jax.ShapeDtypeStruct((B, oh, ow, cout), _BF),
        grid=(B // bb,),
        in_specs=[
            pl.BlockSpec((bb,) + xr.shape[1:], lambda i: (i, 0, 0, 0, 0)),
            pl.BlockSpec(wm.shape, lambda i: (0, 0)),
            pl.BlockSpec((1, cout), lambda i: (0, 0)),
        ],
        out_specs=pl.BlockSpec((bb, oh, ow, cout), lambda i: (i, 0, 0, 0)),
        compiler_params=pltpu.CompilerParams(
            dimension_semantics=("parallel",), vmem_limit_bytes=_VMEM),
    )(xr, wm, b.reshape(1, cout))


def _bandc_body(x_ref, w_ref, b_ref, o_ref, *, oh, pad_out):
    x = x_ref[...]
    bb = x.shape[0]
    k = x.shape[-1]
    n = w_ref.shape[-1]
    acc = b_ref[...].astype(jnp.float32)
    for kh in range(4):
        t = x[:, kh // 2:kh // 2 + oh, kh % 2:kh % 2 + 1, :]
        acc = acc + jnp.dot(t.reshape(bb * oh, k), w_ref[kh],
                            preferred_element_type=jnp.float32)
    acc = jnp.maximum(acc, 0.0)
    y = acc.reshape(bb, oh, n).astype(o_ref.dtype)
    if pad_out:
        z = jnp.zeros((bb, 1, n), o_ref.dtype)
        y = jnp.concatenate([z, y, z], axis=1)
    o_ref[...] = y


def _conv_banded(xr, wb, bias, oh, bb, *, pad_out):
    B = xr.shape[0]
    bb = min(bb, B)
    n = wb.shape[-1]
    orows = oh + 2 if pad_out else oh
    return pl.pallas_call(
        functools.partial(_bandc_body, oh=oh, pad_out=pad_out),
        out_shape=jax.ShapeDtypeStruct((B, orows, n), _BF),
        grid=(B // bb,),
        in_specs=[
            pl.BlockSpec((bb,) + xr.shape[1:], lambda i: (i, 0, 0, 0)),
            pl.BlockSpec(wb.shape, lambda i: (0, 0, 0)),
            pl.BlockSpec((1, n), lambda i: (0, 0)),
        ],
        out_specs=pl.BlockSpec((bb, orows, n), lambda i: (i, 0, 0)),
        compiler_params=pltpu.CompilerParams(
            dimension_semantics=("parallel",), vmem_limit_bytes=_VMEM),
    )(xr, wb, bias.reshape(1, n))


def _band4_body(x_ref, w_ref, b_ref, o_ref):
    x = x_ref[...]
    bb = x.shape[0]
    acc = b_ref[...].astype(jnp.float32)
    for dh in (0, 1):
        t = x[:, dh:dh + 33, :]
        acc = acc + jnp.dot(t.reshape(bb * 33, 1088), w_ref[dh],
                            preferred_element_type=jnp.float32)
    y = acc.reshape(bb, 33, 396)
    for c in range(3):
        y0 = y[:, :, c * 66 + 1:c * 66 + 65]
        y1 = y[:, :, (3 + c) * 66 + 1:(3 + c) * 66 + 65]
        plane = jnp.stack([y1[:, 0:32], y0[:, 1:33]],
                          axis=2).reshape(bb, 64, 64)
        o_ref[:, c, :, :] = plane


def _deconv4_banded(xr, wb, bb_vec, bb=32):
    B = xr.shape[0]
    bb = min(bb, B)
    return pl.pallas_call(
        _band4_body,
        out_shape=jax.ShapeDtypeStruct((B, 3, 64, 64), jnp.float32),
        grid=(B // bb,),
        in_specs=[
            pl.BlockSpec((bb, 34, 1088), lambda i: (i, 0, 0)),
            pl.BlockSpec((2, 1088, 396), lambda i: (0, 0, 0)),
            pl.BlockSpec((1, 396), lambda i: (0, 0)),
        ],
        out_specs=pl.BlockSpec((bb, 3, 64, 64), lambda i: (i, 0, 0, 0)),
        compiler_params=pltpu.CompilerParams(
            dimension_semantics=("parallel",), vmem_limit_bytes=_VMEM),
    )(xr, wb, bb_vec.reshape(1, 396))


def _conv1_nchw_body(x_ref, w_ref, b_ref, o_ref):
    x = x_ref[...].astype(_BF)
    bb = x.shape[0]
    acc = b_ref[...].astype(jnp.float32)
    z1 = jnp.zeros((bb, 1, 64), _BF)
    for c in range(3):
        col0 = x[:, c, :, 0, :]
        col1 = x[:, c, :, 1, :]
        taps = (
            jnp.concatenate([z1, col1[:, 0:31]], axis=1),
            col0,
            col1,
            jnp.concatenate([col0[:, 1:32], z1], axis=1),
        )
        for kh in range(4):
            acc = acc + jnp.dot(taps[kh].reshape(bb * 32, 64),
                                w_ref[kh * 3 + c],
                                preferred_element_type=jnp.float32)
    acc = jnp.maximum(acc, 0.0)
    y = acc.reshape(bb, 32, 1024).astype(o_ref.dtype)
    z = jnp.zeros((bb, 1, 1024), o_ref.dtype)
    o_ref[...] = jnp.concatenate([z, y, z], axis=1)


def _conv1_nchw(x5, wb, bias, bb=32):
    B = x5.shape[0]
    bb = min(bb, B)
    return pl.pallas_call(
        _conv1_nchw_body,
        out_shape=jax.ShapeDtypeStruct((B, 34, 1024), _BF),
        grid=(B // bb,),
        in_specs=[
            pl.BlockSpec((bb, 3, 32, 2, 64), lambda i: (i, 0, 0, 0, 0)),
            pl.BlockSpec((12, 64, 1024), lambda i: (0, 0, 0)),
            pl.BlockSpec((1, 1024), lambda i: (0, 0)),
        ],
        out_specs=pl.BlockSpec((bb, 34, 1024), lambda i: (i, 0, 0)),
        compiler_params=pltpu.CompilerParams(
            dimension_semantics=("parallel",), vmem_limit_bytes=_VMEM),
    )(x5, wb, bias.reshape(1, 1024))


def _band_conv_wm(w, win, ow, *, padded_in):
    cout, cin = w.shape[0], w.shape[1]
    off = 0 if padded_in else 1
    P = np.zeros((win, 4, ow), np.float32)
    for j in range(ow):
        for kw in range(4):
            wi = 2 * j + kw - off
            if 0 <= wi < win:
                P[wi, kw, j] = 1.0
    wr = jnp.transpose(w, (2, 3, 1, 0))
    return (jnp.einsum('wkj,hkcn->hwcjn', jnp.asarray(P), wr)
            .reshape(4, win * cin, ow * cout).astype(_BF))


_P4 = np.zeros((34, 2, 33), np.float32)
for _j in range(33):
    for _dw in range(2):
        _P4[_j + _dw, _dw, _j] = 1.0


def _dec_body(x_ref, w_ref, b_ref, o_ref, *, h, w, relu, shuffle):
    x = x_ref[...]
    taps = [x[:, dh:dh + h + 1, dw:dw + w + 1, :]
            for dh in (0, 1) for dw in (0, 1)]
    a = jnp.concatenate(taps, axis=-1)
    bb = a.shape[0]
    a2 = a.reshape(bb * (h + 1) * (w + 1), a.shape[-1])
    acc = jnp.dot(a2, w_ref[...], preferred_element_type=jnp.float32)
    acc = acc + b_ref[...]
    if relu:
        acc = jnp.maximum(acc, 0.0)
    n4 = acc.shape[-1]
    c = n4 // 4
    if not shuffle:
        o_ref[...] = acc.reshape(bb, h + 1, w + 1, n4).astype(o_ref.dtype)
        return
    y = acc.astype(o_ref.dtype).reshape(bb, h + 1, w + 1, n4)
    y0, y1 = y[..., :2 * c], y[..., 2 * c:]
    r = jnp.stack([y1[:, 0:h], y0[:, 1:h + 1]],
                  axis=2).reshape(bb, 2 * h, w + 1, 2 * c)
    rp0, rp1 = r[..., :c], r[..., c:]
    o_ref[...] = jnp.concatenate(
        [rp1[:, :, 0:w, :], rp0[:, :, 1:w + 1, :]], axis=-1)


def _dec_conv(xp, wm, b, h, w, bb, *, relu, shuffle, out_dtype):
    B = xp.shape[0]
    bb = min(bb, B)
    n4 = wm.shape[1]
    oshape = ((B, 2 * h, w, n4 // 2) if shuffle
              else (B, h + 1, w + 1, n4))
    blk = (bb,) + oshape[1:]
    return pl.pallas_call(
        functools.partial(_dec_body, h=h, w=w, relu=relu, shuffle=shuffle),
        out_shape=jax.ShapeDtypeStruct(oshape, out_dtype),
        grid=(B // bb,),
        in_specs=[
            pl.BlockSpec((bb,) + xp.shape[1:], lambda i: (i, 0, 0, 0)),
            pl.BlockSpec(wm.shape, lambda i: (0, 0)),
            pl.BlockSpec((1, n4), lambda i: (0, 0)),
        ],
        out_specs=pl.BlockSpec(blk, lambda i: (i, 0, 0, 0)),
        compiler_params=pltpu.CompilerParams(
            dimension_semantics=("parallel",), vmem_limit_bytes=_VMEM),
    )(xp, wm, b.reshape(1, n4))


def _latent_body(h_ref, noise_ref, wfc_ref, bfc_ref, wml_ref, bml_ref,
                 wlat_ref, blat_ref, wdec_ref, bdec_ref,
                 d_ref, z_ref, mean_ref, logvar_ref, *, nl):
    hv = h_ref[...]
    acc = bfc_ref[...].astype(jnp.float32)
    for i in range(4):
        acc = acc + jnp.dot(hv[:, i, :], wfc_ref[i],
                            preferred_element_type=jnp.float32)
    h1 = jnp.maximum(acc, 0.0)
    ml = (jnp.dot(h1.astype(_BF), wml_ref[...],
                  preferred_element_type=jnp.float32) + bml_ref[...])
    mean = ml[:, :nl]
    logvar = ml[:, nl:]
    z = noise_ref[...] * jnp.exp(0.5 * logvar) + mean
    d1 = jnp.maximum(
        jnp.dot(z.astype(_BF), wlat_ref[...],
                preferred_element_type=jnp.float32) + blat_ref[...], 0.0)
    d2 = jnp.maximum(
        jnp.dot(d1.astype(_BF), wdec_ref[...],
                preferred_element_type=jnp.float32) + bdec_ref[...], 0.0)
    d_ref[...] = d2.astype(d_ref.dtype)
    z_ref[...] = z
    mean_ref[...] = mean
    logvar_ref[...] = logvar


def _latent(h, noise, wfc, bfc, wml, bml, wlat, blat, wdec, bdec, nl):
    B = h.shape[0]
    bb = B // 2 if B % 2 == 0 else B
    full = lambda arr: pl.BlockSpec(arr.shape, lambda i: (0, 0))
    out_shapes = (
        jax.ShapeDtypeStruct((B, 1024), _BF),
        jax.ShapeDtypeStruct((B, nl), jnp.float32),
        jax.ShapeDtypeStruct((B, nl), jnp.float32),
        jax.ShapeDtypeStruct((B, nl), jnp.float32),
    )
    return pl.pallas_call(
        functools.partial(_latent_body, nl=nl),
        out_shape=out_shapes,
        grid=(B // bb,),
        in_specs=[
            pl.BlockSpec((bb, 4, 256), lambda i: (i, 0, 0)),
            pl.BlockSpec((bb, nl), lambda i: (i, 0)),
            pl.BlockSpec(wfc.shape, lambda i: (0, 0, 0)),
            full(bfc), full(wml), full(bml),
            full(wlat), full(blat), full(wdec), full(bdec),
        ],
        out_specs=(
            pl.BlockSpec((bb, 1024), lambda i: (i, 0)),
            pl.BlockSpec((bb, nl), lambda i: (i, 0)),
            pl.BlockSpec((bb, nl), lambda i: (i, 0)),
            pl.BlockSpec((bb, nl), lambda i: (i, 0)),
        ),
        compiler_params=pltpu.CompilerParams(
            dimension_semantics=("parallel",), vmem_limit_bytes=_VMEM),
    )(h, noise, wfc, bfc, wml, bml, wlat, blat, wdec, bdec)


def _conv_wm(wt):
    return jnp.transpose(wt, (2, 3, 1, 0)).reshape(-1, wt.shape[0]).astype(_BF)


def _phase_wm(wt):
    sel = ((2, 0), (3, 1))
    rows = []
    for dh in range(2):
        for dw in range(2):
            cols = [wt[:, :, sel[ph][dh], sel[pw][dw]]
                    for ph in range(2) for pw in range(2)]
            rows.append(jnp.concatenate(cols, axis=1))
    return jnp.concatenate(rows, axis=0).astype(_BF)


def _pair(y):
    B, H, W, C = y.shape
    yp = jnp.pad(y, ((0, 0), (1, 1), (1, 1), (0, 0)))
    return yp.reshape(B, (H + 2) // 2, 2, (W + 2) // 2, 2 * C)


def _halo(y):
    return jnp.pad(y, ((0, 0), (1, 1), (1, 1), (0, 0)))


def kernel(x, noise,
           conv1_w, conv1_b, conv2_w, conv2_b, conv3_w, conv3_b,
           conv4_w, conv4_b, fc_w, fc_b, fc_mean_w, fc_mean_b,
           fc_logvar_w, fc_logvar_b, fc_latent_w, fc_latent_b,
           fc_dec_w, fc_dec_b, deconv1_w, deconv1_b, deconv2_w, deconv2_b,
           deconv3_w, deconv3_b, deconv4_w, deconv4_b):
    B = x.shape[0]
    nl = noise.shape[1]

    p1u = np.zeros((64, 4, 32), np.float32)
    for j in range(32):
        for kw in range(4):
            wi = 2 * j + kw - 1
            if 0 <= wi < 64:
                p1u[wi, kw, j] = 1.0
    w1r = jnp.transpose(conv1_w, (2, 3, 1, 0))
    wb1 = (jnp.einsum('wkj,hkcn->hcwjn', jnp.asarray(p1u), w1r)
           .reshape(12, 64, 1024).astype(_BF))
    h = _conv1_nchw(x.reshape(B, 3, 32, 2, 64), wb1,
                    jnp.tile(conv1_b, 32))
    h = _conv_banded(h.reshape(B, 17, 2, 1024),
                     _band_conv_wm(conv2_w, 32, 16, padded_in=False),
                     jnp.tile(conv2_b, 16), 16, 32, pad_out=True)
    h = _conv_banded(h.reshape(B, 9, 2, 512),
                     _band_conv_wm(conv3_w, 16, 8, padded_in=False),
                     jnp.tile(conv3_b, 8), 8, 32, pad_out=True)
    h3 = _conv_banded(h.reshape(B, 5, 2, 512),
                      _band_conv_wm(conv4_w, 8, 4, padded_in=False),
                      jnp.tile(conv4_b, 4), 4, 64, pad_out=False)

    wfc_p = (fc_w.T.reshape(64, 4, 4, 256).transpose(1, 2, 0, 3)
             .reshape(4, 256, 256).astype(_BF))
    wml = jnp.concatenate([fc_mean_w.T, fc_logvar_w.T], axis=1).astype(_BF)
    bml = jnp.concatenate([fc_mean_b, fc_logvar_b]).reshape(1, -1)
    wdec_p = (fc_dec_w.T.reshape(256, 64, 4, 4).transpose(0, 2, 3, 1)
              .reshape(256, 1024).astype(_BF))
    bdec_p = fc_dec_b.reshape(64, 4, 4).transpose(1, 2, 0).reshape(1, 1024)
    d, z, z_mean, z_logvar = _latent(
        h3, noise,
        wfc_p, fc_b.reshape(1, -1), wml, bml,
        fc_latent_w.T.astype(_BF), fc_latent_b.reshape(1, -1),
        wdec_p, bdec_p, nl)

    g = d.reshape(B, 4, 4, 64)
    g = _dec_conv(_halo(g), _phase_wm(deconv1_w), jnp.tile(deconv1_b, 4),
                  4, 4, 64, relu=True, shuffle=True, out_dtype=_BF)
    g = _dec_conv(_halo(g.reshape(B, 8, 8, 64)), _phase_wm(deconv2_w),
                  jnp.tile(deconv2_b, 4),
                  8, 8, 32, relu=True, shuffle=True, out_dtype=_BF)
    g = _dec_conv(_halo(g.reshape(B, 16, 16, 32)), _phase_wm(deconv3_w),
                  jnp.tile(deconv3_b, 4),
                  16, 16, 32, relu=True, shuffle=True, out_dtype=_BF)
    x4 = _halo(g.reshape(B, 32, 32, 32)).reshape(B, 34, 1088)
    pw4 = _phase_wm(deconv4_w).astype(jnp.float32).reshape(2, 2, 32, 2, 2, 3)
    wb4 = (jnp.einsum('wdj,hdcpqn->hwcpnjq', jnp.asarray(_P4), pw4)
           .reshape(2, 1088, 396).astype(_BF))
    recon = _deconv4_banded(x4, wb4, jnp.tile(jnp.repeat(deconv4_b, 66), 2))
    return recon, z, z_mean, z_logvar


# speedup vs baseline: 1.0087x; 1.0087x over previous
import functools

import jax
import jax.numpy as jnp
import numpy as np
from jax.experimental import pallas as pl
from jax.experimental.pallas import tpu as pltpu

_BF = jnp.bfloat16
_VMEM = 64 * 1024 * 1024


def _bandc_body(x_ref, w_ref, b_ref, o_ref, *, oh, pad_out):
    x = x_ref[...]
    bb = x.shape[0]
    k = x.shape[-1]
    n = w_ref.shape[-1]
    acc = b_ref[...].astype(jnp.float32)
    for kh in range(4):
        t = x[:, kh // 2:kh // 2 + oh, kh % 2:kh % 2 + 1, :]
        acc = acc + jnp.dot(t.reshape(bb * oh, k), w_ref[kh],
                            preferred_element_type=jnp.float32)
    acc = jnp.maximum(acc, 0.0)
    y = acc.reshape(bb, oh, n).astype(o_ref.dtype)
    if pad_out:
        z = jnp.zeros((bb, 1, n), o_ref.dtype)
        y = jnp.concatenate([z, y, z], axis=1)
    o_ref[...] = y


def _conv_banded(xr, wb, bias, oh, bb, *, pad_out):
    B = xr.shape[0]
    bb = min(bb, B)
    n = wb.shape[-1]
    orows = oh + 2 if pad_out else oh
    return pl.pallas_call(
        functools.partial(_bandc_body, oh=oh, pad_out=pad_out),
        out_shape=jax.ShapeDtypeStruct((B, orows, n), _BF),
        grid=(B // bb,),
        in_specs=[
            pl.BlockSpec((bb,) + xr.shape[1:], lambda i: (i, 0, 0, 0)),
            pl.BlockSpec(wb.shape, lambda i: (0, 0, 0)),
            pl.BlockSpec((1, n), lambda i: (0, 0)),
        ],
        out_specs=pl.BlockSpec((bb, orows, n), lambda i: (i, 0, 0)),
        compiler_params=pltpu.CompilerParams(
            dimension_semantics=("parallel",), vmem_limit_bytes=_VMEM),
    )(xr, wb, bias.reshape(1, n))


def _band4_body(x_ref, w_ref, b_ref, o_ref):
    x = x_ref[...]
    bb = x.shape[0]
    acc = b_ref[...].astype(jnp.float32)
    for dh in (0, 1):
        t = x[:, dh:dh + 33, :]
        acc = acc + jnp.dot(t.reshape(bb * 33, 1088), w_ref[dh],
                            preferred_element_type=jnp.float32)
    y = acc.reshape(bb, 33, 396)
    for c in range(3):
        y0 = y[:, :, c * 66 + 1:c * 66 + 65]
        y1 = y[:, :, (3 + c) * 66 + 1:(3 + c) * 66 + 65]
        plane = jnp.stack([y1[:, 0:32], y0[:, 1:33]],
                          axis=2).reshape(bb, 64, 64)
        o_ref[:, c, :, :] = plane


def _deconv4_banded(xr, wb, bb_vec, bb=32):
    B = xr.shape[0]
    bb = min(bb, B)
    return pl.pallas_call(
        _band4_body,
        out_shape=jax.ShapeDtypeStruct((B, 3, 64, 64), jnp.float32),
        grid=(B // bb,),
        in_specs=[
            pl.BlockSpec((bb, 34, 1088), lambda i: (i, 0, 0)),
            pl.BlockSpec((2, 1088, 396), lambda i: (0, 0, 0)),
            pl.BlockSpec((1, 396), lambda i: (0, 0)),
        ],
        out_specs=pl.BlockSpec((bb, 3, 64, 64), lambda i: (i, 0, 0, 0)),
        compiler_params=pltpu.CompilerParams(
            dimension_semantics=("parallel",), vmem_limit_bytes=_VMEM),
    )(xr, wb, bb_vec.reshape(1, 396))


def _conv1_nchw_body(x_ref, w_ref, b_ref, o_ref):
    x = x_ref[...].astype(_BF)
    bb = x.shape[0]
    acc = b_ref[...].astype(jnp.float32)
    z1 = jnp.zeros((bb, 1, 64), _BF)
    for c in range(3):
        col0 = x[:, c, :, 0, :]
        col1 = x[:, c, :, 1, :]
        taps = (
            jnp.concatenate([z1, col1[:, 0:31]], axis=1),
            col0,
            col1,
            jnp.concatenate([col0[:, 1:32], z1], axis=1),
        )
        for kh in range(4):
            acc = acc + jnp.dot(taps[kh].reshape(bb * 32, 64),
                                w_ref[kh * 3 + c],
                                preferred_element_type=jnp.float32)
    acc = jnp.maximum(acc, 0.0)
    y = acc.reshape(bb, 32, 1024).astype(o_ref.dtype)
    z = jnp.zeros((bb, 1, 1024), o_ref.dtype)
    o_ref[...] = jnp.concatenate([z, y, z], axis=1)


def _conv1_nchw(x5, wb, bias, bb=16):
    B = x5.shape[0]
    bb = min(bb, B)
    return pl.pallas_call(
        _conv1_nchw_body,
        out_shape=jax.ShapeDtypeStruct((B, 34, 1024), _BF),
        grid=(B // bb,),
        in_specs=[
            pl.BlockSpec((bb, 3, 32, 2, 64), lambda i: (i, 0, 0, 0, 0)),
            pl.BlockSpec((12, 64, 1024), lambda i: (0, 0, 0)),
            pl.BlockSpec((1, 1024), lambda i: (0, 0)),
        ],
        out_specs=pl.BlockSpec((bb, 34, 1024), lambda i: (i, 0, 0)),
        compiler_params=pltpu.CompilerParams(
            dimension_semantics=("parallel",), vmem_limit_bytes=_VMEM),
    )(x5, wb, bias.reshape(1, 1024))


def _band_conv_wm(w, win, ow, *, padded_in):
    cout, cin = w.shape[0], w.shape[1]
    off = 0 if padded_in else 1
    P = np.zeros((win, 4, ow), np.float32)
    for j in range(ow):
        for kw in range(4):
            wi = 2 * j + kw - off
            if 0 <= wi < win:
                P[wi, kw, j] = 1.0
    wr = jnp.transpose(w, (2, 3, 1, 0))
    return (jnp.einsum('wkj,hkcn->hwcjn', jnp.asarray(P), wr)
            .reshape(4, win * cin, ow * cout).astype(_BF))


_P4 = np.zeros((34, 2, 33), np.float32)
for _j in range(33):
    for _dw in range(2):
        _P4[_j + _dw, _dw, _j] = 1.0


def _dec_body(x_ref, w_ref, b_ref, o_ref, *, h, w, relu, shuffle):
    x = x_ref[...]
    taps = [x[:, dh:dh + h + 1, dw:dw + w + 1, :]
            for dh in (0, 1) for dw in (0, 1)]
    a = jnp.concatenate(taps, axis=-1)
    bb = a.shape[0]
    a2 = a.reshape(bb * (h + 1) * (w + 1), a.shape[-1])
    acc = jnp.dot(a2, w_ref[...], preferred_element_type=jnp.float32)
    acc = acc + b_ref[...]
    if relu:
        acc = jnp.maximum(acc, 0.0)
    n4 = acc.shape[-1]
    c = n4 // 4
    if not shuffle:
        o_ref[...] = acc.reshape(bb, h + 1, w + 1, n4).astype(o_ref.dtype)
        return
    y = acc.astype(o_ref.dtype).reshape(bb, h + 1, w + 1, n4)
    y0, y1 = y[..., :2 * c], y[..., 2 * c:]
    r = jnp.stack([y1[:, 0:h], y0[:, 1:h + 1]],
                  axis=2).reshape(bb, 2 * h, w + 1, 2 * c)
    rp0, rp1 = r[..., :c], r[..., c:]
    o_ref[...] = jnp.concatenate(
        [rp1[:, :, 0:w, :], rp0[:, :, 1:w + 1, :]], axis=-1)


def _dec_conv(xp, wm, b, h, w, bb, *, relu, shuffle, out_dtype):
    B = xp.shape[0]
    bb = min(bb, B)
    n4 = wm.shape[1]
    oshape = ((B, 2 * h, w, n4 // 2) if shuffle
              else (B, h + 1, w + 1, n4))
    blk = (bb,) + oshape[1:]
    return pl.pallas_call(
        functools.partial(_dec_body, h=h, w=w, relu=relu, shuffle=shuffle),
        out_shape=jax.ShapeDtypeStruct(oshape, out_dtype),
        grid=(B // bb,),
        in_specs=[
            pl.BlockSpec((bb,) + xp.shape[1:], lambda i: (i, 0, 0, 0)),
            pl.BlockSpec(wm.shape, lambda i: (0, 0)),
            pl.BlockSpec((1, n4), lambda i: (0, 0)),
        ],
        out_specs=pl.BlockSpec(blk, lambda i: (i, 0, 0, 0)),
        compiler_params=pltpu.CompilerParams(
            dimension_semantics=("parallel",), vmem_limit_bytes=_VMEM),
    )(xp, wm, b.reshape(1, n4))


def _latent_body(h_ref, noise_ref, wfc_ref, bfc_ref, wml_ref, bml_ref,
                 wlat_ref, blat_ref, wdec_ref, bdec_ref,
                 d_ref, z_ref, mean_ref, logvar_ref, *, nl):
    hv = h_ref[...]
    acc = bfc_ref[...].astype(jnp.float32)
    for i in range(4):
        acc = acc + jnp.dot(hv[:, i, :], wfc_ref[i],
                            preferred_element_type=jnp.float32)
    h1 = jnp.maximum(acc, 0.0)
    ml = (jnp.dot(h1.astype(_BF), wml_ref[...],
                  preferred_element_type=jnp.float32) + bml_ref[...])
    mean = ml[:, :nl]
    logvar = ml[:, nl:]
    z = noise_ref[...] * jnp.exp(0.5 * logvar) + mean
    d1 = jnp.maximum(
        jnp.dot(z.astype(_BF), wlat_ref[...],
                preferred_element_type=jnp.float32) + blat_ref[...], 0.0)
    d2 = jnp.maximum(
        jnp.dot(d1.astype(_BF), wdec_ref[...],
                preferred_element_type=jnp.float32) + bdec_ref[...], 0.0)
    d_ref[...] = d2.astype(d_ref.dtype)
    z_ref[...] = z
    mean_ref[...] = mean
    logvar_ref[...] = logvar


def _latent(h, noise, wfc, bfc, wml, bml, wlat, blat, wdec, bdec, nl):
    B = h.shape[0]
    bb = B // 2 if B % 2 == 0 else B
    full = lambda arr: pl.BlockSpec(arr.shape, lambda i: (0, 0))
    out_shapes = (
        jax.ShapeDtypeStruct((B, 1024), _BF),
        jax.ShapeDtypeStruct((B, nl), jnp.float32),
        jax.ShapeDtypeStruct((B, nl), jnp.float32),
        jax.ShapeDtypeStruct((B, nl), jnp.float32),
    )
    return pl.pallas_call(
        functools.partial(_latent_body, nl=nl),
        out_shape=out_shapes,
        grid=(B // bb,),
        in_specs=[
            pl.BlockSpec((bb, 4, 256), lambda i: (i, 0, 0)),
            pl.BlockSpec((bb, nl), lambda i: (i, 0)),
            pl.BlockSpec(wfc.shape, lambda i: (0, 0, 0)),
            full(bfc), full(wml), full(bml),
            full(wlat), full(blat), full(wdec), full(bdec),
        ],
        out_specs=(
            pl.BlockSpec((bb, 1024), lambda i: (i, 0)),
            pl.BlockSpec((bb, nl), lambda i: (i, 0)),
            pl.BlockSpec((bb, nl), lambda i: (i, 0)),
            pl.BlockSpec((bb, nl), lambda i: (i, 0)),
        ),
        compiler_params=pltpu.CompilerParams(
            dimension_semantics=("parallel",), vmem_limit_bytes=_VMEM),
    )(h, noise, wfc, bfc, wml, bml, wlat, blat, wdec, bdec)


def _phase_wm(wt):
    sel = ((2, 0), (3, 1))
    rows = []
    for dh in range(2):
        for dw in range(2):
            cols = [wt[:, :, sel[ph][dh], sel[pw][dw]]
                    for ph in range(2) for pw in range(2)]
            rows.append(jnp.concatenate(cols, axis=1))
    return jnp.concatenate(rows, axis=0).astype(_BF)


def _halo(y):
    return jnp.pad(y, ((0, 0), (1, 1), (1, 1), (0, 0)))


def kernel(x, noise,
           conv1_w, conv1_b, conv2_w, conv2_b, conv3_w, conv3_b,
           conv4_w, conv4_b, fc_w, fc_b, fc_mean_w, fc_mean_b,
           fc_logvar_w, fc_logvar_b, fc_latent_w, fc_latent_b,
           fc_dec_w, fc_dec_b, deconv1_w, deconv1_b, deconv2_w, deconv2_b,
           deconv3_w, deconv3_b, deconv4_w, deconv4_b):
    B = x.shape[0]
    nl = noise.shape[1]

    p1u = np.zeros((64, 4, 32), np.float32)
    for j in range(32):
        for kw in range(4):
            wi = 2 * j + kw - 1
            if 0 <= wi < 64:
                p1u[wi, kw, j] = 1.0
    w1r = jnp.transpose(conv1_w, (2, 3, 1, 0))
    wb1 = (jnp.einsum('wkj,hkcn->hcwjn', jnp.asarray(p1u), w1r)
           .reshape(12, 64, 1024).astype(_BF))
    h = _conv1_nchw(x.reshape(B, 3, 32, 2, 64), wb1,
                    jnp.tile(conv1_b, 32))
    h = _conv_banded(h.reshape(B, 17, 2, 1024),
                     _band_conv_wm(conv2_w, 32, 16, padded_in=False),
                     jnp.tile(conv2_b, 16), 16, 32, pad_out=True)
    h = _conv_banded(h.reshape(B, 9, 2, 512),
                     _band_conv_wm(conv3_w, 16, 8, padded_in=False),
                     jnp.tile(conv3_b, 8), 8, 32, pad_out=True)
    h3 = _conv_banded(h.reshape(B, 5, 2, 512),
                      _band_conv_wm(conv4_w, 8, 4, padded_in=False),
                      jnp.tile(conv4_b, 4), 4, 64, pad_out=False)

    wfc_p = (fc_w.T.reshape(64, 4, 4, 256).transpose(1, 2, 0, 3)
             .reshape(4, 256, 256).astype(_BF))
    wml = jnp.concatenate([fc_mean_w.T, fc_logvar_w.T], axis=1).astype(_BF)
    bml = jnp.concatenate([fc_mean_b, fc_logvar_b]).reshape(1, -1)
    wdec_p = (fc_dec_w.T.reshape(256, 64, 4, 4).transpose(0, 2, 3, 1)
              .reshape(256, 1024).astype(_BF))
    bdec_p = fc_dec_b.reshape(64, 4, 4).transpose(1, 2, 0).reshape(1, 1024)
    d, z, z_mean, z_logvar = _latent(
        h3, noise,
        wfc_p, fc_b.reshape(1, -1), wml, bml,
        fc_latent_w.T.astype(_BF), fc_latent_b.reshape(1, -1),
        wdec_p, bdec_p, nl)

    g = d.reshape(B, 4, 4, 64)
    g = _dec_conv(_halo(g), _phase_wm(deconv1_w), jnp.tile(deconv1_b, 4),
                  4, 4, 64, relu=True, shuffle=True, out_dtype=_BF)
    g = _dec_conv(_halo(g.reshape(B, 8, 8, 64)), _phase_wm(deconv2_w),
                  jnp.tile(deconv2_b, 4),
                  8, 8, 32, relu=True, shuffle=True, out_dtype=_BF)
    g = _dec_conv(_halo(g.reshape(B, 16, 16, 32)), _phase_wm(deconv3_w),
                  jnp.tile(deconv3_b, 4),
                  16, 16, 16, relu=True, shuffle=True, out_dtype=_BF)
    x4 = _halo(g.reshape(B, 32, 32, 32)).reshape(B, 34, 1088)
    pw4 = _phase_wm(deconv4_w).astype(jnp.float32).reshape(2, 2, 32, 2, 2, 3)
    wb4 = (jnp.einsum('wdj,hdcpqn->hwcpnjq', jnp.asarray(_P4), pw4)
           .reshape(2, 1088, 396).astype(_BF))
    recon = _deconv4_banded(x4, wb4, jnp.tile(jnp.repeat(deconv4_b, 66), 2))
    return recon, z, z_mean, z_logvar


# speedup vs baseline: 1.0675x; 1.0583x over previous
import functools

import jax
import jax.numpy as jnp
import numpy as np
from jax.experimental import pallas as pl
from jax.experimental.pallas import tpu as pltpu

_BF = jnp.bfloat16
_VMEM = 64 * 1024 * 1024


def _bandc_body(x_ref, w_ref, b_ref, o_ref, *, oh, pad_out):
    x = x_ref[...]
    bb = x.shape[0]
    k = x.shape[-1]
    n = w_ref.shape[-1]
    acc = b_ref[...].astype(jnp.float32)
    for kh in range(4):
        t = x[:, kh // 2:kh // 2 + oh, kh % 2:kh % 2 + 1, :]
        acc = acc + jnp.dot(t.reshape(bb * oh, k), w_ref[kh],
                            preferred_element_type=jnp.float32)
    acc = jnp.maximum(acc, 0.0)
    y = acc.reshape(bb, oh, n).astype(o_ref.dtype)
    if pad_out:
        z = jnp.zeros((bb, 1, n), o_ref.dtype)
        y = jnp.concatenate([z, y, z], axis=1)
    o_ref[...] = y


def _conv_banded(xr, wb, bias, oh, bb, *, pad_out):
    B = xr.shape[0]
    bb = min(bb, B)
    n = wb.shape[-1]
    orows = oh + 2 if pad_out else oh
    return pl.pallas_call(
        functools.partial(_bandc_body, oh=oh, pad_out=pad_out),
        out_shape=jax.ShapeDtypeStruct((B, orows, n), _BF),
        grid=(B // bb,),
        in_specs=[
            pl.BlockSpec((bb,) + xr.shape[1:], lambda i: (i, 0, 0, 0)),
            pl.BlockSpec(wb.shape, lambda i: (0, 0, 0)),
            pl.BlockSpec((1, n), lambda i: (0, 0)),
        ],
        out_specs=pl.BlockSpec((bb, orows, n), lambda i: (i, 0, 0)),
        compiler_params=pltpu.CompilerParams(
            dimension_semantics=("parallel",), vmem_limit_bytes=_VMEM),
    )(xr, wb, bias.reshape(1, n))


def _enc_tail_body(x_ref, noise_ref, w2_ref, b2_ref, w3_ref, b3_ref,
                   w4_ref, b4_ref, wfc_ref, bfc_ref, wml_ref, bml_ref,
                   wlat_ref, blat_ref, wdec_ref, bdec_ref,
                   d_ref, z_ref, mean_ref, logvar_ref, *, nl):
    bb = x_ref.shape[0]

    def band(xp, w_r, b_r, oh):
        k = xp.shape[-1]
        n = w_r.shape[-1]
        acc = b_r[...].astype(jnp.float32)
        for kh in range(4):
            t = xp[:, kh // 2:kh // 2 + oh, kh % 2:kh % 2 + 1, :]
            acc = acc + jnp.dot(t.reshape(bb * oh, k), w_r[kh],
                                preferred_element_type=jnp.float32)
        return jnp.maximum(acc, 0.0).reshape(bb, oh, n).astype(_BF)

    def hpad_pair(y):
        n = y.shape[-1]
        zr = jnp.zeros((bb, 1, n), y.dtype)
        yp = jnp.concatenate([zr, y, zr], axis=1)
        return yp.reshape(bb, (y.shape[1] + 2) // 2, 2, n)

    y2 = band(x_ref[...], w2_ref, b2_ref, 16)
    y3 = band(hpad_pair(y2), w3_ref, b3_ref, 8)
    h3 = band(hpad_pair(y3), w4_ref, b4_ref, 4)
    acc = bfc_ref[...].astype(jnp.float32)
    for i in range(4):
        acc = acc + jnp.dot(h3[:, i, :], wfc_ref[i],
                            preferred_element_type=jnp.float32)
    h1 = jnp.maximum(acc, 0.0)
    ml = (jnp.dot(h1.astype(_BF), wml_ref[...],
                  preferred_element_type=jnp.float32) + bml_ref[...])
    mean = ml[:, :nl]
    logvar = ml[:, nl:]
    z = noise_ref[...] * jnp.exp(0.5 * logvar) + mean
    d1 = jnp.maximum(
        jnp.dot(z.astype(_BF), wlat_ref[...],
                preferred_element_type=jnp.float32) + blat_ref[...], 0.0)
    d2 = jnp.maximum(
        jnp.dot(d1.astype(_BF), wdec_ref[...],
                preferred_element_type=jnp.float32) + bdec_ref[...], 0.0)
    d_ref[...] = d2.astype(d_ref.dtype)
    z_ref[...] = z
    mean_ref[...] = mean
    logvar_ref[...] = logvar


def _enc_tail(h, noise, w2, b2, w3, b3, w4, b4,
              wfc, bfc, wml, bml, wlat, blat, wdec, bdec, nl, bb=32):
    B = h.shape[0]
    bb = min(bb, B)
    full2 = lambda arr: pl.BlockSpec(arr.shape, lambda i: (0, 0))
    full3 = lambda arr: pl.BlockSpec(arr.shape, lambda i: (0, 0, 0))
    out_shapes = (
        jax.ShapeDtypeStruct((B, 1024), _BF),
        jax.ShapeDtypeStruct((B, nl), jnp.float32),
        jax.ShapeDtypeStruct((B, nl), jnp.float32),
        jax.ShapeDtypeStruct((B, nl), jnp.float32),
    )
    row = lambda n: pl.BlockSpec((bb, n), lambda i: (i, 0))
    return pl.pallas_call(
        functools.partial(_enc_tail_body, nl=nl),
        out_shape=out_shapes,
        grid=(B // bb,),
        in_specs=[
            pl.BlockSpec((bb, 17, 2, 1024), lambda i: (i, 0, 0, 0)),
            row(nl),
            full3(w2), full2(b2), full3(w3), full2(b3), full3(w4), full2(b4),
            full3(wfc), full2(bfc), full2(wml), full2(bml),
            full2(wlat), full2(blat), full2(wdec), full2(bdec),
        ],
        out_specs=(row(1024), row(nl), row(nl), row(nl)),
        compiler_params=pltpu.CompilerParams(
            dimension_semantics=("parallel",), vmem_limit_bytes=_VMEM),
    )(h, noise, w2, b2, w3, b3, w4, b4,
      wfc, bfc, wml, bml, wlat, blat, wdec, bdec)


def _band4_body(x_ref, w_ref, b_ref, o_ref):
    x = x_ref[...]
    bb = x.shape[0]
    acc = b_ref[...].astype(jnp.float32)
    for dh in (0, 1):
        t = x[:, dh:dh + 33, :]
        acc = acc + jnp.dot(t.reshape(bb * 33, 1088), w_ref[dh],
                            preferred_element_type=jnp.float32)
    y = acc.reshape(bb, 33, 396)
    for c in range(3):
        y0 = y[:, :, c * 66 + 1:c * 66 + 65]
        y1 = y[:, :, (3 + c) * 66 + 1:(3 + c) * 66 + 65]
        plane = jnp.stack([y1[:, 0:32], y0[:, 1:33]],
                          axis=2).reshape(bb, 64, 64)
        o_ref[:, c, :, :] = plane


def _deconv4_banded(xr, wb, bb_vec, bb=32):
    B = xr.shape[0]
    bb = min(bb, B)
    return pl.pallas_call(
        _band4_body,
        out_shape=jax.ShapeDtypeStruct((B, 3, 64, 64), jnp.float32),
        grid=(B // bb,),
        in_specs=[
            pl.BlockSpec((bb, 34, 1088), lambda i: (i, 0, 0)),
            pl.BlockSpec((2, 1088, 396), lambda i: (0, 0, 0)),
            pl.BlockSpec((1, 396), lambda i: (0, 0)),
        ],
        out_specs=pl.BlockSpec((bb, 3, 64, 64), lambda i: (i, 0, 0, 0)),
        compiler_params=pltpu.CompilerParams(
            dimension_semantics=("parallel",), vmem_limit_bytes=_VMEM),
    )(xr, wb, bb_vec.reshape(1, 396))


def _conv1_nchw_body(x_ref, w_ref, b_ref, o_ref):
    x = x_ref[...].astype(_BF)
    bb = x.shape[0]
    acc = b_ref[...].astype(jnp.float32)
    z1 = jnp.zeros((bb, 1, 64), _BF)
    for c in range(3):
        col0 = x[:, c, :, 0, :]
        col1 = x[:, c, :, 1, :]
        taps = (
            jnp.concatenate([z1, col1[:, 0:31]], axis=1),
            col0,
            col1,
            jnp.concatenate([col0[:, 1:32], z1], axis=1),
        )
        for kh in range(4):
            acc = acc + jnp.dot(taps[kh].reshape(bb * 32, 64),
                                w_ref[kh * 3 + c],
                                preferred_element_type=jnp.float32)
    acc = jnp.maximum(acc, 0.0)
    y = acc.reshape(bb, 32, 1024).astype(o_ref.dtype)
    z = jnp.zeros((bb, 1, 1024), o_ref.dtype)
    o_ref[...] = jnp.concatenate([z, y, z], axis=1)


def _conv1_nchw(x5, wb, bias, bb=16):
    B = x5.shape[0]
    bb = min(bb, B)
    return pl.pallas_call(
        _conv1_nchw_body,
        out_shape=jax.ShapeDtypeStruct((B, 34, 1024), _BF),
        grid=(B // bb,),
        in_specs=[
            pl.BlockSpec((bb, 3, 32, 2, 64), lambda i: (i, 0, 0, 0, 0)),
            pl.BlockSpec((12, 64, 1024), lambda i: (0, 0, 0)),
            pl.BlockSpec((1, 1024), lambda i: (0, 0)),
        ],
        out_specs=pl.BlockSpec((bb, 34, 1024), lambda i: (i, 0, 0)),
        compiler_params=pltpu.CompilerParams(
            dimension_semantics=("parallel",), vmem_limit_bytes=_VMEM),
    )(x5, wb, bias.reshape(1, 1024))


def _band_conv_wm(w, win, ow, *, padded_in):
    cout, cin = w.shape[0], w.shape[1]
    off = 0 if padded_in else 1
    P = np.zeros((win, 4, ow), np.float32)
    for j in range(ow):
        for kw in range(4):
            wi = 2 * j + kw - off
            if 0 <= wi < win:
                P[wi, kw, j] = 1.0
    wr = jnp.transpose(w, (2, 3, 1, 0))
    return (jnp.einsum('wkj,hkcn->hwcjn', jnp.asarray(P), wr)
            .reshape(4, win * cin, ow * cout).astype(_BF))


_P4 = np.zeros((34, 2, 33), np.float32)
for _j in range(33):
    for _dw in range(2):
        _P4[_j + _dw, _dw, _j] = 1.0


def _dec_body(x_ref, w_ref, b_ref, o_ref, *, h, w, relu, shuffle):
    x = x_ref[...]
    taps = [x[:, dh:dh + h + 1, dw:dw + w + 1, :]
            for dh in (0, 1) for dw in (0, 1)]
    a = jnp.concatenate(taps, axis=-1)
    bb = a.shape[0]
    a2 = a.reshape(bb * (h + 1) * (w + 1), a.shape[-1])
    acc = jnp.dot(a2, w_ref[...], preferred_element_type=jnp.float32)
    acc = acc + b_ref[...]
    if relu:
        acc = jnp.maximum(acc, 0.0)
    n4 = acc.shape[-1]
    c = n4 // 4
    if not shuffle:
        o_ref[...] = acc.reshape(bb, h + 1, w + 1, n4).astype(o_ref.dtype)
        return
    y = acc.astype(o_ref.dtype).reshape(bb, h + 1, w + 1, n4)
    y0, y1 = y[..., :2 * c], y[..., 2 * c:]
    r = jnp.stack([y1[:, 0:h], y0[:, 1:h + 1]],
                  axis=2).reshape(bb, 2 * h, w + 1, 2 * c)
    rp0, rp1 = r[..., :c], r[..., c:]
    o_ref[...] = jnp.concatenate(
        [rp1[:, :, 0:w, :], rp0[:, :, 1:w + 1, :]], axis=-1)


def _dec_conv(xp, wm, b, h, w, bb, *, relu, shuffle, out_dtype):
    B = xp.shape[0]
    bb = min(bb, B)
    n4 = wm.shape[1]
    oshape = ((B, 2 * h, w, n4 // 2) if shuffle
              else (B, h + 1, w + 1, n4))
    blk = (bb,) + oshape[1:]
    return pl.pallas_call(
        functools.partial(_dec_body, h=h, w=w, relu=relu, shuffle=shuffle),
        out_shape=jax.ShapeDtypeStruct(oshape, out_dtype),
        grid=(B // bb,),
        in_specs=[
            pl.BlockSpec((bb,) + xp.shape[1:], lambda i: (i, 0, 0, 0)),
            pl.BlockSpec(wm.shape, lambda i: (0, 0)),
            pl.BlockSpec((1, n4), lambda i: (0, 0)),
        ],
        out_specs=pl.BlockSpec(blk, lambda i: (i, 0, 0, 0)),
        compiler_params=pltpu.CompilerParams(
            dimension_semantics=("parallel",), vmem_limit_bytes=_VMEM),
    )(xp, wm, b.reshape(1, n4))


def _latent_body(h_ref, noise_ref, wfc_ref, bfc_ref, wml_ref, bml_ref,
                 wlat_ref, blat_ref, wdec_ref, bdec_ref,
                 d_ref, z_ref, mean_ref, logvar_ref, *, nl):
    hv = h_ref[...]
    acc = bfc_ref[...].astype(jnp.float32)
    for i in range(4):
        acc = acc + jnp.dot(hv[:, i, :], wfc_ref[i],
                            preferred_element_type=jnp.float32)
    h1 = jnp.maximum(acc, 0.0)
    ml = (jnp.dot(h1.astype(_BF), wml_ref[...],
                  preferred_element_type=jnp.float32) + bml_ref[...])
    mean = ml[:, :nl]
    logvar = ml[:, nl:]
    z = noise_ref[...] * jnp.exp(0.5 * logvar) + mean
    d1 = jnp.maximum(
        jnp.dot(z.astype(_BF), wlat_ref[...],
                preferred_element_type=jnp.float32) + blat_ref[...], 0.0)
    d2 = jnp.maximum(
        jnp.dot(d1.astype(_BF), wdec_ref[...],
                preferred_element_type=jnp.float32) + bdec_ref[...], 0.0)
    d_ref[...] = d2.astype(d_ref.dtype)
    z_ref[...] = z
    mean_ref[...] = mean
    logvar_ref[...] = logvar


def _latent(h, noise, wfc, bfc, wml, bml, wlat, blat, wdec, bdec, nl):
    B = h.shape[0]
    bb = B // 2 if B % 2 == 0 else B
    full = lambda arr: pl.BlockSpec(arr.shape, lambda i: (0, 0))
    out_shapes = (
        jax.ShapeDtypeStruct((B, 1024), _BF),
        jax.ShapeDtypeStruct((B, nl), jnp.float32),
        jax.ShapeDtypeStruct((B, nl), jnp.float32),
        jax.ShapeDtypeStruct((B, nl), jnp.float32),
    )
    return pl.pallas_call(
        functools.partial(_latent_body, nl=nl),
        out_shape=out_shapes,
        grid=(B // bb,),
        in_specs=[
            pl.BlockSpec((bb, 4, 256), lambda i: (i, 0, 0)),
            pl.BlockSpec((bb, nl), lambda i: (i, 0)),
            pl.BlockSpec(wfc.shape, lambda i: (0, 0, 0)),
            full(bfc), full(wml), full(bml),
            full(wlat), full(blat), full(wdec), full(bdec),
        ],
        out_specs=(
            pl.BlockSpec((bb, 1024), lambda i: (i, 0)),
            pl.BlockSpec((bb, nl), lambda i: (i, 0)),
            pl.BlockSpec((bb, nl), lambda i: (i, 0)),
            pl.BlockSpec((bb, nl), lambda i: (i, 0)),
        ),
        compiler_params=pltpu.CompilerParams(
            dimension_semantics=("parallel",), vmem_limit_bytes=_VMEM),
    )(h, noise, wfc, bfc, wml, bml, wlat, blat, wdec, bdec)


def _phase_wm(wt):
    sel = ((2, 0), (3, 1))
    rows = []
    for dh in range(2):
        for dw in range(2):
            cols = [wt[:, :, sel[ph][dh], sel[pw][dw]]
                    for ph in range(2) for pw in range(2)]
            rows.append(jnp.concatenate(cols, axis=1))
    return jnp.concatenate(rows, axis=0).astype(_BF)


def _halo(y):
    return jnp.pad(y, ((0, 0), (1, 1), (1, 1), (0, 0)))


def kernel(x, noise,
           conv1_w, conv1_b, conv2_w, conv2_b, conv3_w, conv3_b,
           conv4_w, conv4_b, fc_w, fc_b, fc_mean_w, fc_mean_b,
           fc_logvar_w, fc_logvar_b, fc_latent_w, fc_latent_b,
           fc_dec_w, fc_dec_b, deconv1_w, deconv1_b, deconv2_w, deconv2_b,
           deconv3_w, deconv3_b, deconv4_w, deconv4_b):
    B = x.shape[0]
    nl = noise.shape[1]

    p1u = np.zeros((64, 4, 32), np.float32)
    for j in range(32):
        for kw in range(4):
            wi = 2 * j + kw - 1
            if 0 <= wi < 64:
                p1u[wi, kw, j] = 1.0
    w1r = jnp.transpose(conv1_w, (2, 3, 1, 0))
    wb1 = (jnp.einsum('wkj,hkcn->hcwjn', jnp.asarray(p1u), w1r)
           .reshape(12, 64, 1024).astype(_BF))
    h = _conv1_nchw(x.reshape(B, 3, 32, 2, 64), wb1,
                    jnp.tile(conv1_b, 32))
    wfc_p = (fc_w.T.reshape(64, 4, 4, 256).transpose(1, 2, 0, 3)
             .reshape(4, 256, 256).astype(_BF))
    wml = jnp.concatenate([fc_mean_w.T, fc_logvar_w.T], axis=1).astype(_BF)
    bml = jnp.concatenate([fc_mean_b, fc_logvar_b]).reshape(1, -1)
    wdec_p = (fc_dec_w.T.reshape(256, 64, 4, 4).transpose(0, 2, 3, 1)
              .reshape(256, 1024).astype(_BF))
    bdec_p = fc_dec_b.reshape(64, 4, 4).transpose(1, 2, 0).reshape(1, 1024)
    d, z, z_mean, z_logvar = _enc_tail(
        h.reshape(B, 17, 2, 1024), noise,
        _band_conv_wm(conv2_w, 32, 16, padded_in=False),
        jnp.tile(conv2_b, 16).reshape(1, -1),
        _band_conv_wm(conv3_w, 16, 8, padded_in=False),
        jnp.tile(conv3_b, 8).reshape(1, -1),
        _band_conv_wm(conv4_w, 8, 4, padded_in=False),
        jnp.tile(conv4_b, 4).reshape(1, -1),
        wfc_p, fc_b.reshape(1, -1), wml, bml,
        fc_latent_w.T.astype(_BF), fc_latent_b.reshape(1, -1),
        wdec_p, bdec_p, nl)

    g = d.reshape(B, 4, 4, 64)
    g = _dec_conv(_halo(g), _phase_wm(deconv1_w), jnp.tile(deconv1_b, 4),
                  4, 4, 64, relu=True, shuffle=True, out_dtype=_BF)
    g = _dec_conv(_halo(g.reshape(B, 8, 8, 64)), _phase_wm(deconv2_w),
                  jnp.tile(deconv2_b, 4),
                  8, 8, 32, relu=True, shuffle=True, out_dtype=_BF)
    g = _dec_conv(_halo(g.reshape(B, 16, 16, 32)), _phase_wm(deconv3_w),
                  jnp.tile(deconv3_b, 4),
                  16, 16, 16, relu=True, shuffle=True, out_dtype=_BF)
    x4 = _halo(g.reshape(B, 32, 32, 32)).reshape(B, 34, 1088)
    pw4 = _phase_wm(deconv4_w).astype(jnp.float32).reshape(2, 2, 32, 2, 2, 3)
    wb4 = (jnp.einsum('wdj,hdcpqn->hwcpnjq', jnp.asarray(_P4), pw4)
           .reshape(2, 1088, 396).astype(_BF))
    recon = _deconv4_banded(x4, wb4, jnp.tile(jnp.repeat(deconv4_b, 66), 2))
    return recon, z, z_mean, z_logvar
